# Initial kernel scaffold; baseline (speedup 1.0000x reference)
import jax
import jax.numpy as jnp
from jax.experimental import pallas as pl


def kernel(x, pe, edge_index, edge_attr, batch, enc_W, enc_b, in_g, in_b, pe_g, pe_b, l0_gine_eW, l0_gine_eb, l0_gine_nW, l0_gine_nb, l0_attn_iW, l0_attn_ib, l0_attn_oW, l0_attn_ob, l0_mlp_W1, l0_mlp_b1, l0_mlp_W2, l0_mlp_b2, l0_n1_g, l0_n1_b, l0_n2_g, l0_n2_b, l0_n3_g, l0_n3_b, l0_ln_g, l0_ln_b, l1_gine_eW, l1_gine_eb, l1_gine_nW, l1_gine_nb, l1_attn_iW, l1_attn_ib, l1_attn_oW, l1_attn_ob, l1_mlp_W1, l1_mlp_b1, l1_mlp_W2, l1_mlp_b2, l1_n1_g, l1_n1_b, l1_n2_g, l1_n2_b, l1_n3_g, l1_n3_b, l1_ln_g, l1_ln_b, pre_g, pre_b, dec_W1, dec_b1, dec_W2, dec_b2):
    raise NotImplementedError("write your pallas kernel here")


# speedup vs baseline: 1.5832x; 1.5832x over previous
import functools
import numpy as np
import jax
import jax.numpy as jnp
from jax import lax
from jax.experimental import pallas as pl
from jax.experimental.pallas import tpu as pltpu
from jax.experimental.pallas import tpu_sc as plsc

N = 10000
E = 320000
HID = 128
PE = 16
HENC = HID - PE
HEADS = 4
HD = HID // HEADS
NG = 20
EPS = 1e-5

R = 1000
NR = N // R
TQ = 400
NQ = N // TQ

NC = 2
NS = 16
L = 16
NW = NC * NS
QD = 2560
EPW = E // NS
EB = 80
NB = EPW // EB
RPD = QD // NS

EBLK = 8000
NEB = E // EBLK


def _leaky(x):
    return jnp.where(x >= 0, x, 0.01 * x)


def _row(v):
    return v.reshape(1, -1)


def _acc_stats(i, y, s_ref, q_ref):
    @pl.when(i == 0)
    def _():
        s_ref[...] = jnp.zeros_like(s_ref)
        q_ref[...] = jnp.zeros_like(q_ref)

    s = jnp.sum(y, axis=0, keepdims=True)
    q = jnp.sum(y * y, axis=0, keepdims=True)
    s_ref[...] += jnp.broadcast_to(s, s_ref.shape)
    q_ref[...] += jnp.broadcast_to(q, q_ref.shape)


def _affine(s, q, g, b):
    mu = s / N
    var = q / N - mu * mu
    a = g / jnp.sqrt(var + EPS)
    return a, b - mu * a


def _stats(s, q):
    mu = s / N
    return mu, q / N - mu * mu


def _enc_call(x, pe, W, b):
    def body(x_ref, pe_ref, W_ref, b_ref, h_ref, s1, q1, s2, q2):
        i = pl.program_id(0)
        h = _leaky(jnp.dot(x_ref[...], W_ref[...],
                           preferred_element_type=jnp.float32) + b_ref[...])
        h_ref[...] = h
        _acc_stats(i, h, s1, q1)
        _acc_stats(i, pe_ref[...], s2, q2)

    return pl.pallas_call(
        body,
        grid=(NR,),
        in_specs=[
            pl.BlockSpec((R, HID), lambda i: (i, 0)),
            pl.BlockSpec((R, PE), lambda i: (i, 0)),
            pl.BlockSpec((HID, HENC), lambda i: (0, 0)),
            pl.BlockSpec((1, HENC), lambda i: (0, 0)),
        ],
        out_specs=[
            pl.BlockSpec((R, HENC), lambda i: (i, 0)),
            pl.BlockSpec((8, HENC), lambda i: (0, 0)),
            pl.BlockSpec((8, HENC), lambda i: (0, 0)),
            pl.BlockSpec((8, PE), lambda i: (0, 0)),
            pl.BlockSpec((8, PE), lambda i: (0, 0)),
        ],
        out_shape=[
            jax.ShapeDtypeStruct((N, HENC), jnp.float32),
            jax.ShapeDtypeStruct((8, HENC), jnp.float32),
            jax.ShapeDtypeStruct((8, HENC), jnp.float32),
            jax.ShapeDtypeStruct((8, PE), jnp.float32),
            jax.ShapeDtypeStruct((8, PE), jnp.float32),
        ],
    )(x, pe, W, b)


def _concat_call(henc, pe, a1, c1, a2, c2):
    def body(h_ref, pe_ref, a1r, c1r, a2r, c2r, out_ref):
        hn = h_ref[...] * a1r[...] + c1r[...]
        pn = pe_ref[...] * a2r[...] + c2r[...]
        out_ref[...] = jnp.concatenate([hn, pn], axis=1)

    return pl.pallas_call(
        body,
        grid=(NR,),
        in_specs=[
            pl.BlockSpec((R, HENC), lambda i: (i, 0)),
            pl.BlockSpec((R, PE), lambda i: (i, 0)),
            pl.BlockSpec((1, HENC), lambda i: (0, 0)),
            pl.BlockSpec((1, HENC), lambda i: (0, 0)),
            pl.BlockSpec((1, PE), lambda i: (0, 0)),
            pl.BlockSpec((1, PE), lambda i: (0, 0)),
        ],
        out_specs=pl.BlockSpec((R, HID), lambda i: (i, 0)),
        out_shape=jax.ShapeDtypeStruct((N, HID), jnp.float32),
    )(henc, pe, a1, c1, a2, c2)


def _edge_call(edge_attr, W, b):
    def body(ea_ref, W_ref, b_ref, out_ref):
        out_ref[...] = jnp.dot(ea_ref[...], W_ref[...],
                               preferred_element_type=jnp.float32) + b_ref[...]

    return pl.pallas_call(
        body,
        grid=(NEB,),
        in_specs=[
            pl.BlockSpec((EBLK, PE), lambda i: (i, 0)),
            pl.BlockSpec((PE, HID), lambda i: (0, 0)),
            pl.BlockSpec((1, HID), lambda i: (0, 0)),
        ],
        out_specs=pl.BlockSpec((EBLK, HID), lambda i: (i, 0)),
        out_shape=jax.ShapeDtypeStruct((E, HID), jnp.float32),
    )(edge_attr, W, b)


def _gine_sc(h, e_pre, src_r, dst_r):
    mesh = plsc.VectorSubcoreMesh(core_axis_name="c", subcore_axis_name="s",
                                  num_cores=NC, num_subcores=NS)

    @functools.partial(
        pl.kernel,
        out_type=jax.ShapeDtypeStruct((4, QD, HID), jnp.float32),
        mesh=mesh,
        scratch_types=[
            pltpu.VMEM((EB,), jnp.int32),
            pltpu.VMEM((EB,), jnp.int32),
            pltpu.VMEM((EB,), jnp.int32),
            pltpu.VMEM((EB, HID), jnp.float32),
            pltpu.VMEM((EB, HID), jnp.float32),
            pltpu.VMEM((EB, HID), jnp.float32),
            pltpu.VMEM_SHARED((QD + 8, HID), jnp.float32),
            pltpu.SemaphoreType.DMA,
            pltpu.SemaphoreType.DMA,
        ],
    )
    def k(h_hbm, e_hbm, src_hbm, dst_hbm, out_hbm,
          src_v, dst_v, dstl_v, rows_v, e_v, zbuf, agg_sh, sem1, sem2):
        c = lax.axis_index("c")
        s = lax.axis_index("s")
        base = s * EPW

        def zfill(i, carry):
            def zcol(j, cc):
                zbuf[i, pl.ds(j * L, L)] = jnp.zeros((L,), jnp.float32)
                return cc
            return lax.fori_loop(0, HID // L, zcol, carry, unroll=True)
        lax.fori_loop(0, EB, zfill, 0)

        for p in range(2):
            qr = p * NC + c
            brow = qr * QD

            for t in range(2):
                pltpu.sync_copy(zbuf,
                                agg_sh.at[pl.ds(s * RPD + t * EB, EB), :])

            @pl.when(s == 0)
            def _():
                pltpu.sync_copy(zbuf.at[pl.ds(0, 8), :],
                                agg_sh.at[pl.ds(QD, 8), :])
            plsc.subcore_barrier()

            def blk(j, carry):
                pltpu.sync_copy(src_hbm.at[s].at[j], src_v)
                g1 = pltpu.async_copy(h_hbm.at[src_v], rows_v, sem1)
                g2 = pltpu.async_copy(e_hbm.at[pl.ds(base + j * EB, EB), :],
                                      e_v, sem2)
                pltpu.sync_copy(dst_hbm.at[s].at[j], dst_v)
                def remap(q, cc):
                    t = dst_v[pl.ds(q * L, L)] - brow
                    ok = (t >= 0) & (t < QD)
                    dstl_v[pl.ds(q * L, L)] = jnp.where(ok, t, QD)
                    return cc
                lax.fori_loop(0, EB // L, remap, 0, unroll=True)
                g1.wait()
                g2.wait()

                def crow(r, cc):
                    for q in range(HID // L):
                        v = (rows_v[r, pl.ds(q * L, L)]
                             + e_v[r, pl.ds(q * L, L)])
                        rows_v[r, pl.ds(q * L, L)] = jnp.maximum(v, 0.0)
                    return cc
                lax.fori_loop(0, EB, crow, 0)

                pltpu.sync_copy(rows_v, agg_sh.at[dstl_v], add=True)
                return carry
            lax.fori_loop(0, NB, blk, 0)

            plsc.subcore_barrier()
            pltpu.sync_copy(agg_sh.at[pl.ds(s * RPD, RPD), :],
                            out_hbm.at[qr].at[pl.ds(s * RPD, RPD), :])
            plsc.subcore_barrier()

    return k(h, e_pre, src_r, dst_r)


def _local_call(agg, h, W, b):
    def body(p_ref, hr, Wr, br, y_ref, s_ref, q_ref):
        i = pl.program_id(0)
        hh = hr[...]
        aggh = p_ref[...] + hh
        local = _leaky(jnp.dot(aggh, Wr[...],
                               preferred_element_type=jnp.float32) + br[...])
        y = local + hh
        y_ref[...] = y
        _acc_stats(i, y, s_ref, q_ref)

    return pl.pallas_call(
        body,
        grid=(NR,),
        in_specs=[
            pl.BlockSpec((R, HID), lambda i: (i, 0)),
            pl.BlockSpec((R, HID), lambda i: (i, 0)),
            pl.BlockSpec((HID, HID), lambda i: (0, 0)),
            pl.BlockSpec((1, HID), lambda i: (0, 0)),
        ],
        out_specs=[
            pl.BlockSpec((R, HID), lambda i: (i, 0)),
            pl.BlockSpec((8, HID), lambda i: (0, 0)),
            pl.BlockSpec((8, HID), lambda i: (0, 0)),
        ],
        out_shape=[
            jax.ShapeDtypeStruct((N, HID), jnp.float32),
            jax.ShapeDtypeStruct((8, HID), jnp.float32),
            jax.ShapeDtypeStruct((8, HID), jnp.float32),
        ],
    )(agg, h, W, b)


def _qkv_call(h, Wq, bq, Wk, bk, Wv, bv):
    def body(h_ref, Wq_r, bq_r, Wk_r, bk_r, Wv_r, bv_r, q_ref, k_ref, v_ref):
        hh = h_ref[...]
        q_ref[...] = jnp.dot(hh, Wq_r[...],
                             preferred_element_type=jnp.float32) + bq_r[...]
        k_ref[...] = jnp.dot(hh, Wk_r[...],
                             preferred_element_type=jnp.float32) + bk_r[...]
        v_ref[...] = jnp.dot(hh, Wv_r[...],
                             preferred_element_type=jnp.float32) + bv_r[...]

    wspec = pl.BlockSpec((HID, HID), lambda i: (0, 0))
    bspec = pl.BlockSpec((1, HID), lambda i: (0, 0))
    nspec = pl.BlockSpec((R, HID), lambda i: (i, 0))
    return pl.pallas_call(
        body,
        grid=(NR,),
        in_specs=[nspec, wspec, bspec, wspec, bspec, wspec, bspec],
        out_specs=[nspec, nspec, nspec],
        out_shape=[jax.ShapeDtypeStruct((N, HID), jnp.float32)] * 3,
    )(h, Wq, bq, Wk, bk, Wv, bv)


def _attn_call(ktlo, kthi, q, k, v, boh, h, oW, ob):
    def body(ktlo_ref, kthi_ref, q_ref, k_ref, v_ref, bq_ref, bk_ref, h_ref,
             oW_ref, ob_ref, o_ref, s_ref, sq_ref):
        t = pl.program_id(0)
        lo = ktlo_ref[t]
        hi = kthi_ref[t]
        bq_oh = bq_ref[...]
        scale = np.float32(1.0 / np.sqrt(HD))

        outs = []
        for hh in range(HEADS):
            qh = q_ref[:, hh * HD:(hh + 1) * HD] * scale

            def kbody(kt, carry):
                m, l, acc = carry
                ks = k_ref[pl.ds(kt * TQ, TQ), hh * HD:(hh + 1) * HD]
                vs = v_ref[pl.ds(kt * TQ, TQ), hh * HD:(hh + 1) * HD]
                bk_oh = bk_ref[pl.ds(kt * TQ, TQ), :]
                same = lax.dot_general(bq_oh, bk_oh, (((1,), (1,)), ((), ())),
                                       preferred_element_type=jnp.float32)
                sc = lax.dot_general(qh, ks, (((1,), (1,)), ((), ())),
                                     preferred_element_type=jnp.float32)
                sc = jnp.where(same > 0.5, sc, -1e9)
                mnew = jnp.maximum(m, jnp.max(sc, axis=1, keepdims=True))
                p = jnp.exp(sc - mnew)
                alpha = jnp.exp(m - mnew)
                lnew = l * alpha + jnp.sum(p, axis=1, keepdims=True)
                accnew = acc * alpha + jnp.dot(
                    p, vs, preferred_element_type=jnp.float32)
                return mnew, lnew, accnew

            m0 = jnp.full((TQ, 1), -1e30, jnp.float32)
            l0 = jnp.zeros((TQ, 1), jnp.float32)
            a0 = jnp.zeros((TQ, HD), jnp.float32)
            m, l, acc = lax.fori_loop(lo, hi + 1, kbody, (m0, l0, a0))
            outs.append(acc / l)

        ao = jnp.concatenate(outs, axis=1)
        y = jnp.dot(ao, oW_ref[...],
                    preferred_element_type=jnp.float32) + ob_ref[...] + h_ref[...]
        o_ref[...] = y
        _acc_stats(t, y, s_ref, sq_ref)

    grid_spec = pltpu.PrefetchScalarGridSpec(
        num_scalar_prefetch=2,
        grid=(NQ,),
        in_specs=[
            pl.BlockSpec((TQ, HID), lambda t, lo, hi: (t, 0)),
            pl.BlockSpec((N, HID), lambda t, lo, hi: (0, 0)),
            pl.BlockSpec((N, HID), lambda t, lo, hi: (0, 0)),
            pl.BlockSpec((TQ, 32), lambda t, lo, hi: (t, 0)),
            pl.BlockSpec((N, 32), lambda t, lo, hi: (0, 0)),
            pl.BlockSpec((TQ, HID), lambda t, lo, hi: (t, 0)),
            pl.BlockSpec((HID, HID), lambda t, lo, hi: (0, 0)),
            pl.BlockSpec((1, HID), lambda t, lo, hi: (0, 0)),
        ],
        out_specs=[
            pl.BlockSpec((TQ, HID), lambda t, lo, hi: (t, 0)),
            pl.BlockSpec((8, HID), lambda t, lo, hi: (0, 0)),
            pl.BlockSpec((8, HID), lambda t, lo, hi: (0, 0)),
        ],
    )
    return pl.pallas_call(
        body,
        grid_spec=grid_spec,
        out_shape=[
            jax.ShapeDtypeStruct((N, HID), jnp.float32),
            jax.ShapeDtypeStruct((8, HID), jnp.float32),
            jax.ShapeDtypeStruct((8, HID), jnp.float32),
        ],
    )(ktlo, kthi, q, k, v, boh, boh, h, oW, ob)


def _mlp_call(y1, h2, a1, c1, a2, c2, W1, b1, W2, b2):
    def body(y1r, h2r, a1r, c1r, a2r, c2r, W1r, b1r, W2r, b2r,
             out_ref, s_ref, q_ref):
        i = pl.program_id(0)
        out = y1r[...] * a1r[...] + c1r[...] + h2r[...] * a2r[...] + c2r[...]
        t = jnp.maximum(jnp.dot(out, W1r[...],
                                preferred_element_type=jnp.float32) + b1r[...],
                        0.0)
        m = jnp.dot(t, W2r[...], preferred_element_type=jnp.float32) + b2r[...]
        o2 = out + m
        out_ref[...] = o2
        _acc_stats(i, o2, s_ref, q_ref)

    nspec = pl.BlockSpec((R, HID), lambda i: (i, 0))
    bspec = pl.BlockSpec((1, HID), lambda i: (0, 0))
    return pl.pallas_call(
        body,
        grid=(NR,),
        in_specs=[
            nspec, nspec, bspec, bspec, bspec, bspec,
            pl.BlockSpec((HID, 2 * HID), lambda i: (0, 0)),
            pl.BlockSpec((1, 2 * HID), lambda i: (0, 0)),
            pl.BlockSpec((2 * HID, HID), lambda i: (0, 0)),
            bspec,
        ],
        out_specs=[
            nspec,
            pl.BlockSpec((8, HID), lambda i: (0, 0)),
            pl.BlockSpec((8, HID), lambda i: (0, 0)),
        ],
        out_shape=[
            jax.ShapeDtypeStruct((N, HID), jnp.float32),
            jax.ShapeDtypeStruct((8, HID), jnp.float32),
            jax.ShapeDtypeStruct((8, HID), jnp.float32),
        ],
    )(y1, h2, a1, c1, a2, c2, W1, b1, W2, b2)


def _affine_apply_call(x, a, c):
    def body(x_ref, a_ref, c_ref, out_ref):
        out_ref[...] = x_ref[...] * a_ref[...] + c_ref[...]

    nspec = pl.BlockSpec((R, HID), lambda i: (i, 0))
    bspec = pl.BlockSpec((1, HID), lambda i: (0, 0))
    return pl.pallas_call(
        body,
        grid=(NR,),
        in_specs=[nspec, bspec, bspec],
        out_specs=nspec,
        out_shape=jax.ShapeDtypeStruct((N, HID), jnp.float32),
    )(x, a, c)


def _dec_call(x, a, c, W1, b1, W2, b2):
    def body(x_ref, a_ref, c_ref, W1r, b1r, W2r, b2r, out_ref):
        hp = x_ref[...] * a_ref[...] + c_ref[...]
        d = _leaky(jnp.dot(hp, W1r[...],
                           preferred_element_type=jnp.float32) + b1r[...])
        out_ref[...] = jnp.dot(d, W2r[...],
                               preferred_element_type=jnp.float32) + b2r[...]

    nspec = pl.BlockSpec((R, HID), lambda i: (i, 0))
    bspec = pl.BlockSpec((1, HID), lambda i: (0, 0))
    wspec = pl.BlockSpec((HID, HID), lambda i: (0, 0))
    return pl.pallas_call(
        body,
        grid=(NR,),
        in_specs=[nspec, bspec, bspec, wspec, bspec, wspec, bspec],
        out_specs=nspec,
        out_shape=jax.ShapeDtypeStruct((N, HID), jnp.float32),
    )(x, a, c, W1, b1, W2, b2)


def _gps_layer(h, edge_attr, src_r, dst_r, ktlo, kthi, boh, P):
    e_pre = _edge_call(edge_attr, P["eW"], _row(P["eb"]))
    parts = _gine_sc(h, e_pre, src_r, dst_r)
    agg = parts.reshape(4 * QD, HID)[:N]
    y1, s1, q1 = _local_call(agg, h, P["nW"], _row(P["nb"]))

    iW, ib = P["iW"], P["ib"]
    qm, km, vm = _qkv_call(
        h,
        iW[:, :HID], _row(ib[:HID]),
        iW[:, HID:2 * HID], _row(ib[HID:2 * HID]),
        iW[:, 2 * HID:], _row(ib[2 * HID:]))
    h2, s2, q2 = _attn_call(ktlo, kthi, qm, km, vm, boh, h,
                            P["oW"], _row(P["ob"]))

    a1, c1 = _affine(s1[0], q1[0], P["n1_g"], P["n1_b"])
    a2, c2 = _affine(s2[0], q2[0], P["n2_g"], P["n2_b"])
    out2, s3, q3 = _mlp_call(y1, h2, _row(a1), _row(c1), _row(a2), _row(c2),
                             P["W1"], _row(P["b1"]), P["W2"], _row(P["b2"]))
    return out2, s3[0], q3[0]


def _fold_n3_ln(s3, q3, g3, b3, g_ln, b_ln):
    mu2, var2 = _stats(s3, q3)
    s3d = jnp.sqrt(var2 + EPS)
    v_ln = g3 * g3 * var2 / (var2 + EPS)
    kaff = g3 * g_ln / (s3d * jnp.sqrt(v_ln + EPS))
    return kaff, b_ln - mu2 * kaff, mu2, var2


def kernel(x, pe, edge_index, edge_attr, batch, enc_W, enc_b, in_g, in_b,
           pe_g, pe_b,
           l0_gine_eW, l0_gine_eb, l0_gine_nW, l0_gine_nb,
           l0_attn_iW, l0_attn_ib, l0_attn_oW, l0_attn_ob,
           l0_mlp_W1, l0_mlp_b1, l0_mlp_W2, l0_mlp_b2,
           l0_n1_g, l0_n1_b, l0_n2_g, l0_n2_b, l0_n3_g, l0_n3_b,
           l0_ln_g, l0_ln_b,
           l1_gine_eW, l1_gine_eb, l1_gine_nW, l1_gine_nb,
           l1_attn_iW, l1_attn_ib, l1_attn_oW, l1_attn_ob,
           l1_mlp_W1, l1_mlp_b1, l1_mlp_W2, l1_mlp_b2,
           l1_n1_g, l1_n1_b, l1_n2_g, l1_n2_b, l1_n3_g, l1_n3_b,
           l1_ln_g, l1_ln_b,
           pre_g, pre_b, dec_W1, dec_b1, dec_W2, dec_b2):
    src_r = edge_index[0].reshape(NS, NB, EB)
    dst_r = edge_index[1].reshape(NS, NB, EB)
    boh = jax.nn.one_hot(batch, 32, dtype=jnp.float32)
    gids = jnp.arange(NG, dtype=jnp.int32)
    starts = jnp.searchsorted(batch, gids, side="left").astype(jnp.int32)
    ends = jnp.searchsorted(batch, gids, side="right").astype(jnp.int32)
    tstart = jnp.arange(NQ, dtype=jnp.int32) * TQ
    g_lo = batch[tstart]
    g_hi = batch[tstart + TQ - 1]
    ktlo = (starts[g_lo] // TQ).astype(jnp.int32)
    kthi = ((ends[g_hi] - 1) // TQ).astype(jnp.int32)

    henc, s1, q1, s2, q2 = _enc_call(x, pe, enc_W, _row(enc_b))
    a1, c1 = _affine(s1[0], q1[0], in_g, in_b)
    a2, c2 = _affine(s2[0], q2[0], pe_g, pe_b)
    h = _concat_call(henc, pe, _row(a1), _row(c1), _row(a2), _row(c2))

    P0 = dict(eW=l0_gine_eW, eb=l0_gine_eb, nW=l0_gine_nW, nb=l0_gine_nb,
              iW=l0_attn_iW, ib=l0_attn_ib, oW=l0_attn_oW, ob=l0_attn_ob,
              W1=l0_mlp_W1, b1=l0_mlp_b1, W2=l0_mlp_W2, b2=l0_mlp_b2,
              n1_g=l0_n1_g, n1_b=l0_n1_b, n2_g=l0_n2_g, n2_b=l0_n2_b)
    out2, s3, q3 = _gps_layer(h, edge_attr, src_r, dst_r, ktlo, kthi,
                              boh, P0)
    A, C, _, _ = _fold_n3_ln(s3, q3, l0_n3_g, l0_n3_b, l0_ln_g, l0_ln_b)
    h = _affine_apply_call(out2, _row(A), _row(C))

    P1 = dict(eW=l1_gine_eW, eb=l1_gine_eb, nW=l1_gine_nW, nb=l1_gine_nb,
              iW=l1_attn_iW, ib=l1_attn_ib, oW=l1_attn_oW, ob=l1_attn_ob,
              W1=l1_mlp_W1, b1=l1_mlp_b1, W2=l1_mlp_W2, b2=l1_mlp_b2,
              n1_g=l1_n1_g, n1_b=l1_n1_b, n2_g=l1_n2_g, n2_b=l1_n2_b)
    out2, s3, q3 = _gps_layer(h, edge_attr, src_r, dst_r, ktlo, kthi,
                              boh, P1)

    kaff, _, mu2, var2 = _fold_n3_ln(s3, q3, l1_n3_g, l1_n3_b,
                                     l1_ln_g, l1_ln_b)
    var_h = kaff * kaff * var2
    A2 = kaff * pre_g / jnp.sqrt(var_h + EPS)
    C2 = pre_b - mu2 * A2
    return _dec_call(out2, _row(A2), _row(C2), dec_W1, _row(dec_b1),
                     dec_W2, _row(dec_b2))


# speedup vs baseline: 2.4226x; 1.5301x over previous
import functools
import numpy as np
import jax
import jax.numpy as jnp
from jax import lax
from jax.experimental import pallas as pl
from jax.experimental.pallas import tpu as pltpu
from jax.experimental.pallas import tpu_sc as plsc

N = 10000
E = 320000
HID = 128
PE = 16
HENC = HID - PE
HEADS = 4
HD = HID // HEADS
NG = 20
EPS = 1e-5

R = 1000
NR = N // R
TQ = 400
NQ = N // TQ

NC = 2
NS = 16
L = 16
NW = NC * NS
QD = 2560
EPW = E // NS
EB = 80
NB = EPW // EB
RPD = QD // NS
CHK = 10

EBLK = 8000
NEB = E // EBLK


def _leaky(x):
    return jnp.where(x >= 0, x, 0.01 * x)


def _row(v):
    return v.reshape(1, -1)


def _acc_stats(i, y, s_ref, q_ref):
    @pl.when(i == 0)
    def _():
        s_ref[...] = jnp.zeros_like(s_ref)
        q_ref[...] = jnp.zeros_like(q_ref)

    s = jnp.sum(y, axis=0, keepdims=True)
    q = jnp.sum(y * y, axis=0, keepdims=True)
    s_ref[...] += jnp.broadcast_to(s, s_ref.shape)
    q_ref[...] += jnp.broadcast_to(q, q_ref.shape)


def _affine(s, q, g, b):
    mu = s / N
    var = q / N - mu * mu
    a = g / jnp.sqrt(var + EPS)
    return a, b - mu * a


def _stats(s, q):
    mu = s / N
    return mu, q / N - mu * mu


def _enc_call(x, pe, W, b):
    def body(x_ref, pe_ref, W_ref, b_ref, h_ref, s1, q1, s2, q2):
        i = pl.program_id(0)
        h = _leaky(jnp.dot(x_ref[...], W_ref[...],
                           preferred_element_type=jnp.float32) + b_ref[...])
        h_ref[...] = h
        _acc_stats(i, h, s1, q1)
        _acc_stats(i, pe_ref[...], s2, q2)

    return pl.pallas_call(
        body,
        grid=(NR,),
        in_specs=[
            pl.BlockSpec((R, HID), lambda i: (i, 0)),
            pl.BlockSpec((R, PE), lambda i: (i, 0)),
            pl.BlockSpec((HID, HENC), lambda i: (0, 0)),
            pl.BlockSpec((1, HENC), lambda i: (0, 0)),
        ],
        out_specs=[
            pl.BlockSpec((R, HENC), lambda i: (i, 0)),
            pl.BlockSpec((8, HENC), lambda i: (0, 0)),
            pl.BlockSpec((8, HENC), lambda i: (0, 0)),
            pl.BlockSpec((8, PE), lambda i: (0, 0)),
            pl.BlockSpec((8, PE), lambda i: (0, 0)),
        ],
        out_shape=[
            jax.ShapeDtypeStruct((N, HENC), jnp.float32),
            jax.ShapeDtypeStruct((8, HENC), jnp.float32),
            jax.ShapeDtypeStruct((8, HENC), jnp.float32),
            jax.ShapeDtypeStruct((8, PE), jnp.float32),
            jax.ShapeDtypeStruct((8, PE), jnp.float32),
        ],
    )(x, pe, W, b)


def _concat_call(henc, pe, a1, c1, a2, c2):
    def body(h_ref, pe_ref, a1r, c1r, a2r, c2r, out_ref):
        hn = h_ref[...] * a1r[...] + c1r[...]
        pn = pe_ref[...] * a2r[...] + c2r[...]
        out_ref[...] = jnp.concatenate([hn, pn], axis=1)

    return pl.pallas_call(
        body,
        grid=(NR,),
        in_specs=[
            pl.BlockSpec((R, HENC), lambda i: (i, 0)),
            pl.BlockSpec((R, PE), lambda i: (i, 0)),
            pl.BlockSpec((1, HENC), lambda i: (0, 0)),
            pl.BlockSpec((1, HENC), lambda i: (0, 0)),
            pl.BlockSpec((1, PE), lambda i: (0, 0)),
            pl.BlockSpec((1, PE), lambda i: (0, 0)),
        ],
        out_specs=pl.BlockSpec((R, HID), lambda i: (i, 0)),
        out_shape=jax.ShapeDtypeStruct((N, HID), jnp.float32),
    )(henc, pe, a1, c1, a2, c2)


def _edge_call(edge_attr, W, b):
    def body(ea_ref, W_ref, b_ref, out_ref):
        out_ref[...] = jnp.dot(ea_ref[...], W_ref[...],
                               preferred_element_type=jnp.float32) + b_ref[...]

    return pl.pallas_call(
        body,
        grid=(NEB,),
        in_specs=[
            pl.BlockSpec((EBLK, PE), lambda i: (i, 0)),
            pl.BlockSpec((PE, HID), lambda i: (0, 0)),
            pl.BlockSpec((1, HID), lambda i: (0, 0)),
        ],
        out_specs=pl.BlockSpec((EBLK, HID), lambda i: (i, 0)),
        out_shape=jax.ShapeDtypeStruct((E, HID), jnp.float32),
    )(edge_attr, W, b)


def _gine_sc(h, e_pre, src_r, dst_r):
    mesh = plsc.VectorSubcoreMesh(core_axis_name="c", subcore_axis_name="s",
                                  num_cores=NC, num_subcores=NS)

    @functools.partial(
        pl.kernel,
        out_type=jax.ShapeDtypeStruct((4, QD, HID), jnp.float32),
        mesh=mesh,
        scratch_types=[
            pltpu.VMEM((CHK * EB,), jnp.int32),
            pltpu.VMEM((CHK * EB,), jnp.int32),
            [pltpu.VMEM((EB,), jnp.int32) for _ in range(2)],
            [pltpu.VMEM((EB, HID), jnp.float32) for _ in range(2)],
            [pltpu.VMEM((EB, HID), jnp.float32) for _ in range(2)],
            [pltpu.VMEM((EB, HID), jnp.float32) for _ in range(2)],
            pltpu.VMEM((EB, HID), jnp.float32),
            pltpu.VMEM_SHARED((QD + 8, HID), jnp.float32),
            [pltpu.SemaphoreType.DMA for _ in range(2)],
            [pltpu.SemaphoreType.DMA for _ in range(2)],
            [pltpu.SemaphoreType.DMA for _ in range(2)],
        ],
    )
    def k(h_hbm, e_hbm, src_hbm, dst_hbm, out_hbm,
          srcC, dstC, dstl_v, rows_v, e_v, msg_v, zbuf, agg_sh,
          g_sem, e_sem, sc_sem):
        c = lax.axis_index("c")
        s = lax.axis_index("s")
        base = s * EPW

        def zfill(i, carry):
            def zcol(j, cc):
                zbuf[i, pl.ds(j * L, L)] = jnp.zeros((L,), jnp.float32)
                return cc
            return lax.fori_loop(0, HID // L, zcol, carry, unroll=True)
        lax.fori_loop(0, EB, zfill, 0)

        def start_ge(j, u, b):
            idx = srcC.at[pl.ds(u * EB, EB)]
            pltpu.async_copy(h_hbm.at[idx], rows_v[b], g_sem[b])
            pltpu.async_copy(e_hbm.at[pl.ds(base + j * EB, EB), :],
                             e_v[b], e_sem[b])

        def wait_ge(b):
            pltpu.make_async_copy(h_hbm.at[srcC.at[pl.ds(0, EB)]],
                                  rows_v[b], g_sem[b]).wait()
            pltpu.make_async_copy(e_hbm.at[pl.ds(0, EB), :],
                                  e_v[b], e_sem[b]).wait()

        def wait_sc(b):
            pltpu.make_async_copy(msg_v[b], agg_sh.at[dstl_v[b]],
                                  sc_sem[b]).wait()

        def run_pass(p):
            qr = p * NC + c
            brow = qr * QD

            for t in range(2):
                pltpu.sync_copy(zbuf,
                                agg_sh.at[pl.ds(s * RPD + t * EB, EB), :])

            @pl.when(s == 0)
            def _():
                pltpu.sync_copy(zbuf.at[pl.ds(0, 8), :],
                                agg_sh.at[pl.ds(QD, 8), :])
            plsc.subcore_barrier()

            def chunk(ch, carry):
                j0 = ch * CHK
                pltpu.sync_copy(
                    src_hbm.at[pl.ds(base + j0 * EB, CHK * EB)], srcC)
                pltpu.sync_copy(
                    dst_hbm.at[pl.ds(base + j0 * EB, CHK * EB)], dstC)
                start_ge(j0, 0, 0)
                for u in range(CHK):
                    j = j0 + u
                    b = u % 2
                    if u + 1 < CHK:
                        start_ge(j + 1, u + 1, 1 - b)
                    wait_ge(b)
                    if u >= 2:
                        wait_sc(b)
                    else:
                        @pl.when(ch >= 1)
                        def _():
                            wait_sc(b)
                    def remap(q, cc):
                        t2 = dstC[pl.ds(u * EB + q * L, L)] - brow
                        ok = (t2 >= 0) & (t2 < QD)
                        dstl_v[b][pl.ds(q * L, L)] = jnp.where(ok, t2, QD)
                        return cc
                    lax.fori_loop(0, EB // L, remap, 0, unroll=True)

                    def crow(r, cc):
                        for q in range(HID // L):
                            v = (rows_v[b][r, pl.ds(q * L, L)]
                                 + e_v[b][r, pl.ds(q * L, L)])
                            msg_v[b][r, pl.ds(q * L, L)] = jnp.maximum(v, 0.0)
                        return cc
                    lax.fori_loop(0, EB, crow, 0)

                    pltpu.async_copy(msg_v[b], agg_sh.at[dstl_v[b]],
                                     sc_sem[b], add=True)
                return carry
            lax.fori_loop(0, NB // CHK, chunk, 0)
            wait_sc(0)
            wait_sc(1)

            plsc.subcore_barrier()
            pltpu.sync_copy(agg_sh.at[pl.ds(s * RPD, RPD), :],
                            out_hbm.at[qr].at[pl.ds(s * RPD, RPD), :])
            plsc.subcore_barrier()

        for p in range(2):
            run_pass(p)

    return k(h, e_pre, src_r, dst_r)


def _local_call(agg, h, W, b):
    def body(p_ref, hr, Wr, br, y_ref, s_ref, q_ref):
        i = pl.program_id(0)
        hh = hr[...]
        aggh = p_ref[...] + hh
        local = _leaky(jnp.dot(aggh, Wr[...],
                               preferred_element_type=jnp.float32) + br[...])
        y = local + hh
        y_ref[...] = y
        _acc_stats(i, y, s_ref, q_ref)

    return pl.pallas_call(
        body,
        grid=(NR,),
        in_specs=[
            pl.BlockSpec((R, HID), lambda i: (i, 0)),
            pl.BlockSpec((R, HID), lambda i: (i, 0)),
            pl.BlockSpec((HID, HID), lambda i: (0, 0)),
            pl.BlockSpec((1, HID), lambda i: (0, 0)),
        ],
        out_specs=[
            pl.BlockSpec((R, HID), lambda i: (i, 0)),
            pl.BlockSpec((8, HID), lambda i: (0, 0)),
            pl.BlockSpec((8, HID), lambda i: (0, 0)),
        ],
        out_shape=[
            jax.ShapeDtypeStruct((N, HID), jnp.float32),
            jax.ShapeDtypeStruct((8, HID), jnp.float32),
            jax.ShapeDtypeStruct((8, HID), jnp.float32),
        ],
    )(agg, h, W, b)


def _qkv_call(h, Wq, bq, Wk, bk, Wv, bv):
    def body(h_ref, Wq_r, bq_r, Wk_r, bk_r, Wv_r, bv_r, q_ref, k_ref, v_ref):
        hh = h_ref[...]
        q_ref[...] = jnp.dot(hh, Wq_r[...],
                             preferred_element_type=jnp.float32) + bq_r[...]
        k_ref[...] = jnp.dot(hh, Wk_r[...],
                             preferred_element_type=jnp.float32) + bk_r[...]
        v_ref[...] = jnp.dot(hh, Wv_r[...],
                             preferred_element_type=jnp.float32) + bv_r[...]

    wspec = pl.BlockSpec((HID, HID), lambda i: (0, 0))
    bspec = pl.BlockSpec((1, HID), lambda i: (0, 0))
    nspec = pl.BlockSpec((R, HID), lambda i: (i, 0))
    return pl.pallas_call(
        body,
        grid=(NR,),
        in_specs=[nspec, wspec, bspec, wspec, bspec, wspec, bspec],
        out_specs=[nspec, nspec, nspec],
        out_shape=[jax.ShapeDtypeStruct((N, HID), jnp.float32)] * 3,
    )(h, Wq, bq, Wk, bk, Wv, bv)


def _attn_call(ktlo, kthi, q, k, v, boh, h, oW, ob):
    def body(ktlo_ref, kthi_ref, q_ref, k_ref, v_ref, bq_ref, bk_ref, h_ref,
             oW_ref, ob_ref, o_ref, s_ref, sq_ref):
        t = pl.program_id(0)
        lo = ktlo_ref[t]
        hi = kthi_ref[t]
        bq_oh = bq_ref[...]
        scale = np.float32(1.0 / np.sqrt(HD))

        outs = []
        for hh in range(HEADS):
            qh = q_ref[:, hh * HD:(hh + 1) * HD] * scale

            def kbody(kt, carry):
                m, l, acc = carry
                ks = k_ref[pl.ds(kt * TQ, TQ), hh * HD:(hh + 1) * HD]
                vs = v_ref[pl.ds(kt * TQ, TQ), hh * HD:(hh + 1) * HD]
                bk_oh = bk_ref[pl.ds(kt * TQ, TQ), :]
                same = lax.dot_general(bq_oh, bk_oh, (((1,), (1,)), ((), ())),
                                       preferred_element_type=jnp.float32)
                sc = lax.dot_general(qh, ks, (((1,), (1,)), ((), ())),
                                     preferred_element_type=jnp.float32)
                sc = jnp.where(same > 0.5, sc, -1e9)
                mnew = jnp.maximum(m, jnp.max(sc, axis=1, keepdims=True))
                p = jnp.exp(sc - mnew)
                alpha = jnp.exp(m - mnew)
                lnew = l * alpha + jnp.sum(p, axis=1, keepdims=True)
                accnew = acc * alpha + jnp.dot(
                    p, vs, preferred_element_type=jnp.float32)
                return mnew, lnew, accnew

            m0 = jnp.full((TQ, 1), -1e30, jnp.float32)
            l0 = jnp.zeros((TQ, 1), jnp.float32)
            a0 = jnp.zeros((TQ, HD), jnp.float32)
            m, l, acc = lax.fori_loop(lo, hi + 1, kbody, (m0, l0, a0))
            outs.append(acc / l)

        ao = jnp.concatenate(outs, axis=1)
        y = jnp.dot(ao, oW_ref[...],
                    preferred_element_type=jnp.float32) + ob_ref[...] + h_ref[...]
        o_ref[...] = y
        _acc_stats(t, y, s_ref, sq_ref)

    grid_spec = pltpu.PrefetchScalarGridSpec(
        num_scalar_prefetch=2,
        grid=(NQ,),
        in_specs=[
            pl.BlockSpec((TQ, HID), lambda t, lo, hi: (t, 0)),
            pl.BlockSpec((N, HID), lambda t, lo, hi: (0, 0)),
            pl.BlockSpec((N, HID), lambda t, lo, hi: (0, 0)),
            pl.BlockSpec((TQ, 32), lambda t, lo, hi: (t, 0)),
            pl.BlockSpec((N, 32), lambda t, lo, hi: (0, 0)),
            pl.BlockSpec((TQ, HID), lambda t, lo, hi: (t, 0)),
            pl.BlockSpec((HID, HID), lambda t, lo, hi: (0, 0)),
            pl.BlockSpec((1, HID), lambda t, lo, hi: (0, 0)),
        ],
        out_specs=[
            pl.BlockSpec((TQ, HID), lambda t, lo, hi: (t, 0)),
            pl.BlockSpec((8, HID), lambda t, lo, hi: (0, 0)),
            pl.BlockSpec((8, HID), lambda t, lo, hi: (0, 0)),
        ],
    )
    return pl.pallas_call(
        body,
        grid_spec=grid_spec,
        out_shape=[
            jax.ShapeDtypeStruct((N, HID), jnp.float32),
            jax.ShapeDtypeStruct((8, HID), jnp.float32),
            jax.ShapeDtypeStruct((8, HID), jnp.float32),
        ],
    )(ktlo, kthi, q, k, v, boh, boh, h, oW, ob)


def _mlp_call(y1, h2, a1, c1, a2, c2, W1, b1, W2, b2):
    def body(y1r, h2r, a1r, c1r, a2r, c2r, W1r, b1r, W2r, b2r,
             out_ref, s_ref, q_ref):
        i = pl.program_id(0)
        out = y1r[...] * a1r[...] + c1r[...] + h2r[...] * a2r[...] + c2r[...]
        t = jnp.maximum(jnp.dot(out, W1r[...],
                                preferred_element_type=jnp.float32) + b1r[...],
                        0.0)
        m = jnp.dot(t, W2r[...], preferred_element_type=jnp.float32) + b2r[...]
        o2 = out + m
        out_ref[...] = o2
        _acc_stats(i, o2, s_ref, q_ref)

    nspec = pl.BlockSpec((R, HID), lambda i: (i, 0))
    bspec = pl.BlockSpec((1, HID), lambda i: (0, 0))
    return pl.pallas_call(
        body,
        grid=(NR,),
        in_specs=[
            nspec, nspec, bspec, bspec, bspec, bspec,
            pl.BlockSpec((HID, 2 * HID), lambda i: (0, 0)),
            pl.BlockSpec((1, 2 * HID), lambda i: (0, 0)),
            pl.BlockSpec((2 * HID, HID), lambda i: (0, 0)),
            bspec,
        ],
        out_specs=[
            nspec,
            pl.BlockSpec((8, HID), lambda i: (0, 0)),
            pl.BlockSpec((8, HID), lambda i: (0, 0)),
        ],
        out_shape=[
            jax.ShapeDtypeStruct((N, HID), jnp.float32),
            jax.ShapeDtypeStruct((8, HID), jnp.float32),
            jax.ShapeDtypeStruct((8, HID), jnp.float32),
        ],
    )(y1, h2, a1, c1, a2, c2, W1, b1, W2, b2)


def _affine_apply_call(x, a, c):
    def body(x_ref, a_ref, c_ref, out_ref):
        out_ref[...] = x_ref[...] * a_ref[...] + c_ref[...]

    nspec = pl.BlockSpec((R, HID), lambda i: (i, 0))
    bspec = pl.BlockSpec((1, HID), lambda i: (0, 0))
    return pl.pallas_call(
        body,
        grid=(NR,),
        in_specs=[nspec, bspec, bspec],
        out_specs=nspec,
        out_shape=jax.ShapeDtypeStruct((N, HID), jnp.float32),
    )(x, a, c)


def _dec_call(x, a, c, W1, b1, W2, b2):
    def body(x_ref, a_ref, c_ref, W1r, b1r, W2r, b2r, out_ref):
        hp = x_ref[...] * a_ref[...] + c_ref[...]
        d = _leaky(jnp.dot(hp, W1r[...],
                           preferred_element_type=jnp.float32) + b1r[...])
        out_ref[...] = jnp.dot(d, W2r[...],
                               preferred_element_type=jnp.float32) + b2r[...]

    nspec = pl.BlockSpec((R, HID), lambda i: (i, 0))
    bspec = pl.BlockSpec((1, HID), lambda i: (0, 0))
    wspec = pl.BlockSpec((HID, HID), lambda i: (0, 0))
    return pl.pallas_call(
        body,
        grid=(NR,),
        in_specs=[nspec, bspec, bspec, wspec, bspec, wspec, bspec],
        out_specs=nspec,
        out_shape=jax.ShapeDtypeStruct((N, HID), jnp.float32),
    )(x, a, c, W1, b1, W2, b2)


def _gps_layer(h, edge_attr, src_r, dst_r, ktlo, kthi, boh, P):
    e_pre = _edge_call(edge_attr, P["eW"], _row(P["eb"]))
    parts = _gine_sc(h, e_pre, src_r, dst_r)
    agg = parts.reshape(4 * QD, HID)[:N]
    y1, s1, q1 = _local_call(agg, h, P["nW"], _row(P["nb"]))

    iW, ib = P["iW"], P["ib"]
    qm, km, vm = _qkv_call(
        h,
        iW[:, :HID], _row(ib[:HID]),
        iW[:, HID:2 * HID], _row(ib[HID:2 * HID]),
        iW[:, 2 * HID:], _row(ib[2 * HID:]))
    h2, s2, q2 = _attn_call(ktlo, kthi, qm, km, vm, boh, h,
                            P["oW"], _row(P["ob"]))

    a1, c1 = _affine(s1[0], q1[0], P["n1_g"], P["n1_b"])
    a2, c2 = _affine(s2[0], q2[0], P["n2_g"], P["n2_b"])
    out2, s3, q3 = _mlp_call(y1, h2, _row(a1), _row(c1), _row(a2), _row(c2),
                             P["W1"], _row(P["b1"]), P["W2"], _row(P["b2"]))
    return out2, s3[0], q3[0]


def _fold_n3_ln(s3, q3, g3, b3, g_ln, b_ln):
    mu2, var2 = _stats(s3, q3)
    s3d = jnp.sqrt(var2 + EPS)
    v_ln = g3 * g3 * var2 / (var2 + EPS)
    kaff = g3 * g_ln / (s3d * jnp.sqrt(v_ln + EPS))
    return kaff, b_ln - mu2 * kaff, mu2, var2


def kernel(x, pe, edge_index, edge_attr, batch, enc_W, enc_b, in_g, in_b,
           pe_g, pe_b,
           l0_gine_eW, l0_gine_eb, l0_gine_nW, l0_gine_nb,
           l0_attn_iW, l0_attn_ib, l0_attn_oW, l0_attn_ob,
           l0_mlp_W1, l0_mlp_b1, l0_mlp_W2, l0_mlp_b2,
           l0_n1_g, l0_n1_b, l0_n2_g, l0_n2_b, l0_n3_g, l0_n3_b,
           l0_ln_g, l0_ln_b,
           l1_gine_eW, l1_gine_eb, l1_gine_nW, l1_gine_nb,
           l1_attn_iW, l1_attn_ib, l1_attn_oW, l1_attn_ob,
           l1_mlp_W1, l1_mlp_b1, l1_mlp_W2, l1_mlp_b2,
           l1_n1_g, l1_n1_b, l1_n2_g, l1_n2_b, l1_n3_g, l1_n3_b,
           l1_ln_g, l1_ln_b,
           pre_g, pre_b, dec_W1, dec_b1, dec_W2, dec_b2):
    src_r = edge_index[0]
    dst_r = edge_index[1]
    boh = jax.nn.one_hot(batch, 32, dtype=jnp.float32)
    gids = jnp.arange(NG, dtype=jnp.int32)
    starts = jnp.searchsorted(batch, gids, side="left").astype(jnp.int32)
    ends = jnp.searchsorted(batch, gids, side="right").astype(jnp.int32)
    tstart = jnp.arange(NQ, dtype=jnp.int32) * TQ
    g_lo = batch[tstart]
    g_hi = batch[tstart + TQ - 1]
    ktlo = (starts[g_lo] // TQ).astype(jnp.int32)
    kthi = ((ends[g_hi] - 1) // TQ).astype(jnp.int32)

    henc, s1, q1, s2, q2 = _enc_call(x, pe, enc_W, _row(enc_b))
    a1, c1 = _affine(s1[0], q1[0], in_g, in_b)
    a2, c2 = _affine(s2[0], q2[0], pe_g, pe_b)
    h = _concat_call(henc, pe, _row(a1), _row(c1), _row(a2), _row(c2))

    P0 = dict(eW=l0_gine_eW, eb=l0_gine_eb, nW=l0_gine_nW, nb=l0_gine_nb,
              iW=l0_attn_iW, ib=l0_attn_ib, oW=l0_attn_oW, ob=l0_attn_ob,
              W1=l0_mlp_W1, b1=l0_mlp_b1, W2=l0_mlp_W2, b2=l0_mlp_b2,
              n1_g=l0_n1_g, n1_b=l0_n1_b, n2_g=l0_n2_g, n2_b=l0_n2_b)
    out2, s3, q3 = _gps_layer(h, edge_attr, src_r, dst_r, ktlo, kthi,
                              boh, P0)
    A, C, _, _ = _fold_n3_ln(s3, q3, l0_n3_g, l0_n3_b, l0_ln_g, l0_ln_b)
    h = _affine_apply_call(out2, _row(A), _row(C))

    P1 = dict(eW=l1_gine_eW, eb=l1_gine_eb, nW=l1_gine_nW, nb=l1_gine_nb,
              iW=l1_attn_iW, ib=l1_attn_ib, oW=l1_attn_oW, ob=l1_attn_ob,
              W1=l1_mlp_W1, b1=l1_mlp_b1, W2=l1_mlp_W2, b2=l1_mlp_b2,
              n1_g=l1_n1_g, n1_b=l1_n1_b, n2_g=l1_n2_g, n2_b=l1_n2_b)
    out2, s3, q3 = _gps_layer(h, edge_attr, src_r, dst_r, ktlo, kthi,
                              boh, P1)

    kaff, _, mu2, var2 = _fold_n3_ln(s3, q3, l1_n3_g, l1_n3_b,
                                     l1_ln_g, l1_ln_b)
    var_h = kaff * kaff * var2
    A2 = kaff * pre_g / jnp.sqrt(var_h + EPS)
    C2 = pre_b - mu2 * A2
    return _dec_call(out2, _row(A2), _row(C2), dec_W1, _row(dec_b1),
                     dec_W2, _row(dec_b2))
